# Initial kernel scaffold; baseline (speedup 1.0000x reference)
import jax
import jax.numpy as jnp
from jax.experimental import pallas as pl


def kernel(x, edge_index, W, b):
    raise NotImplementedError("write your pallas kernel here")


# speedup vs baseline: 9.3768x; 9.3768x over previous
import functools

import jax
import jax.numpy as jnp
from jax import lax
from jax.experimental import pallas as pl
from jax.experimental.pallas import tpu as pltpu
from jax.experimental.pallas import tpu_sc as plsc

C = 128
N = 10000
K = 32
L = 16

NC, NS = 2, 16
NW = NC * NS
NPT = 320
N_PAD = NW * NPT
G = 4
RG = G * K
NG = NPT // G
NBUF = 2

BN = 1024


def _mm_body(x_ref, w_ref, o_ref):
    acc = lax.dot_general(
        x_ref[...], w_ref[...],
        (((0,), (1,)), ((), ())),
        preferred_element_type=jnp.float32,
    )
    o_ref[...] = jnp.maximum(acc, 0.0)


def _mlp_table(xs_pad, W):
    return pl.pallas_call(
        _mm_body,
        grid=(N_PAD // BN,),
        in_specs=[
            pl.BlockSpec((C, BN), lambda i: (0, i)),
            pl.BlockSpec((C, C), lambda i: (0, 0)),
        ],
        out_specs=pl.BlockSpec((BN, C), lambda i: (i, 0)),
        out_shape=jax.ShapeDtypeStruct((N_PAD, C), jnp.float32),
    )(xs_pad, W)


def _sc_body(h_hbm, idx_hbm, b_hbm, out_hbm,
             idx_v, buf0, buf1, out_v, b_v, sem0, sem1):
    cid = lax.axis_index("c")
    sid = lax.axis_index("s")
    wid = sid * NC + cid

    pltpu.sync_copy(idx_hbm.at[pl.ds(wid * NG, NG)], idx_v)
    pltpu.sync_copy(b_hbm, b_v)

    bufs = (buf0, buf1)
    sems = (sem0, sem1)

    for b in range(NBUF):
        pltpu.make_async_copy(h_hbm.at[idx_v.at[b]], bufs[b], sems[b]).start()

    def iter_body(i, carry):
        for b in range(NBUF):
            g = NBUF * i + b
            buf = bufs[b]
            sem = sems[b]
            pltpu.make_async_copy(h_hbm.at[idx_v.at[g]], buf, sem).wait()
            for j in range(G):
                for c in range(C // L):
                    sl = pl.ds(c * L, L)
                    acc = buf[j * K, sl]
                    for k in range(1, K):
                        acc = jnp.maximum(acc, buf[j * K + k, sl])
                    out_v[g * G + j, sl] = acc * (1.0 / K) + b_v[sl]
            nxt = g + NBUF

            @pl.when(nxt < NG)
            def _():
                pltpu.make_async_copy(
                    h_hbm.at[idx_v.at[nxt]], buf, sem).start()
        return carry

    lax.fori_loop(0, NG // NBUF, iter_body, 0)

    pltpu.sync_copy(out_v, out_hbm.at[pl.ds(wid * NPT, NPT)])


_sc_aggregate = pl.kernel(
    _sc_body,
    out_type=jax.ShapeDtypeStruct((N_PAD, C), jnp.float32),
    mesh=plsc.VectorSubcoreMesh(
        core_axis_name="c", subcore_axis_name="s",
        num_cores=NC, num_subcores=NS),
    scratch_types=[
        pltpu.VMEM((NG, RG), jnp.int32),
        pltpu.VMEM((RG, C), jnp.float32),
        pltpu.VMEM((RG, C), jnp.float32),
        pltpu.VMEM((NPT, C), jnp.float32),
        pltpu.VMEM((C,), jnp.float32),
        pltpu.SemaphoreType.DMA,
        pltpu.SemaphoreType.DMA,
    ],
)


def kernel(x, edge_index, W, b):
    xs = x[0, :, :, 0]
    xs_pad = jnp.pad(xs, ((0, 0), (0, N_PAD - N)))
    h = _mlp_table(xs_pad, W)

    idx = edge_index[0, 0].reshape(-1)
    idx_pad = jnp.pad(idx, (0, N_PAD * K - N * K))
    idx_pad = idx_pad.reshape(NW * NG, RG)

    bvec = b[0, :, 0, 0]

    out_t = _sc_aggregate(h, idx_pad, bvec)
    out = out_t[:N].T[None, :, :, None]
    return out


# speedup vs baseline: 9.4048x; 1.0030x over previous
import jax
import jax.numpy as jnp
from jax import lax
from jax.experimental import pallas as pl
from jax.experimental.pallas import tpu as pltpu
from jax.experimental.pallas import tpu_sc as plsc

C = 128
N = 10000
K = 32
LB = 16

NC, NS = 2, 16
NW = NC * NS
NPT = 320
N_PAD = NW * NPT
G = 4
RG = G * K
NG = NPT // G
NBUF = 4

BN = 1024


def _mm_body(x_ref, w_ref, b_ref, o_ref):
    acc = lax.dot_general(
        x_ref[...], w_ref[...],
        (((0,), (1,)), ((), ())),
        preferred_element_type=jnp.float32,
    )
    h = jnp.maximum(acc, 0.0) * (1.0 / K) + b_ref[...]
    o_ref[...] = h


def _mlp_table(xs_pad, W, bvec):
    return pl.pallas_call(
        _mm_body,
        grid=(N_PAD // BN,),
        in_specs=[
            pl.BlockSpec((C, BN), lambda i: (0, i)),
            pl.BlockSpec((C, C), lambda i: (0, 0)),
            pl.BlockSpec((1, C), lambda i: (0, 0)),
        ],
        out_specs=pl.BlockSpec((BN, C), lambda i: (i, 0)),
        out_shape=jax.ShapeDtypeStruct((N_PAD, C), jnp.float32),
    )(xs_pad, W, bvec)


def _tree_max(vals):
    while len(vals) > 1:
        nxt = [jnp.maximum(vals[2 * t], vals[2 * t + 1])
               for t in range(len(vals) // 2)]
        if len(vals) % 2:
            nxt.append(vals[-1])
        vals = nxt
    return vals[0]


def _sc_body(h_hbm, idx_hbm, out_hbm,
             idx_v, buf0, buf1, buf2, buf3, out_v,
             sem0, sem1, sem2, sem3):
    cid = lax.axis_index("c")
    sid = lax.axis_index("s")
    wid = sid * NC + cid

    pltpu.sync_copy(idx_hbm.at[pl.ds(wid * NG, NG)], idx_v)

    bufs = (buf0, buf1, buf2, buf3)
    sems = (sem0, sem1, sem2, sem3)

    for b in range(NBUF):
        pltpu.make_async_copy(h_hbm.at[idx_v.at[b]], bufs[b], sems[b]).start()

    def iter_body(i, carry):
        for b in range(NBUF):
            g = NBUF * i + b
            buf = bufs[b]
            sem = sems[b]
            pltpu.make_async_copy(h_hbm.at[idx_v.at[g]], buf, sem).wait()
            for j in range(G):
                for c in range(C // LB):
                    sl = pl.ds(c * LB, LB)
                    out_v[g, j, sl] = _tree_max(
                        [buf[j * K + k, sl] for k in range(K)])
            nxt = g + NBUF

            @pl.when(nxt < NG)
            def _():
                pltpu.make_async_copy(
                    h_hbm.at[idx_v.at[nxt]], buf, sem).start()
        return carry

    lax.fori_loop(0, NG // NBUF, iter_body, 0)

    pltpu.sync_copy(out_v, out_hbm.at[pl.ds(wid * NG, NG)])


_sc_aggregate = pl.kernel(
    _sc_body,
    out_type=jax.ShapeDtypeStruct((NW * NG, G, C), jnp.float32),
    mesh=plsc.VectorSubcoreMesh(
        core_axis_name="c", subcore_axis_name="s",
        num_cores=NC, num_subcores=NS),
    scratch_types=[
        pltpu.VMEM((NG, RG), jnp.int32),
        pltpu.VMEM((RG, C), jnp.float32),
        pltpu.VMEM((RG, C), jnp.float32),
        pltpu.VMEM((RG, C), jnp.float32),
        pltpu.VMEM((RG, C), jnp.float32),
        pltpu.VMEM((NG, G, C), jnp.float32),
        pltpu.SemaphoreType.DMA,
        pltpu.SemaphoreType.DMA,
        pltpu.SemaphoreType.DMA,
        pltpu.SemaphoreType.DMA,
    ],
)


def kernel(x, edge_index, W, b):
    xs = x[0, :, :, 0]
    xs_pad = jnp.pad(xs, ((0, 0), (0, N_PAD - N)))
    bvec = b[0, :, 0, 0].reshape(1, C)
    h = _mlp_table(xs_pad, W, bvec)

    idx = edge_index[0, 0].reshape(-1)
    idx_pad = jnp.pad(idx, (0, N_PAD * K - N * K))
    idx_pad = idx_pad.reshape(NW * NG, RG)

    out_t = _sc_aggregate(h, idx_pad)
    out = out_t.reshape(N_PAD, C)[:N].T[None, :, :, None]
    return out
